# baseline (speedup 1.0000x reference)
import functools

import jax
import jax.numpy as jnp
from jax import lax
from jax.experimental import pallas as pl
from jax.experimental.pallas import tpu as pltpu
from jax.experimental.pallas import tpu_sc as plsc

NH = 64
RH = 32
RL = 2
IC = 3
K = 1024
D = 64
CC = 0.25
B = 8
H1 = 112
H2 = 56
M2 = H2 * H2
NPIX = B * M2
VQ_BLK = 1568
VQ_STEPS = NPIX // VQ_BLK
SC_NW = 32
SC_CHUNK = NPIX // SC_NW

_HIGH = jax.lax.Precision.DEFAULT
_PAR = pltpu.CompilerParams(dimension_semantics=("parallel",))


def _mm(a, b):
    return jax.lax.dot_general(a, b, (((1,), (0,)), ((), ())),
                               precision=_HIGH,
                               preferred_element_type=jnp.float32)


def _pad1(h3):
    H, W, C = h3.shape
    zc = jnp.zeros((H, 1, C), h3.dtype)
    hp = jnp.concatenate([zc, h3, zc], axis=1)
    zr = jnp.zeros((1, W + 2, C), h3.dtype)
    return jnp.concatenate([zr, hp, zr], axis=0)


def _conv3(h3, w9, b):
    H, W, C = h3.shape
    O = w9.shape[2]
    hp = _pad1(h3)
    acc = jnp.zeros((H * W, O), jnp.float32)
    t = 0
    for di in range(3):
        for dj in range(3):
            acc = acc + _mm(hp[di:di + H, dj:dj + W, :].reshape(H * W, C), w9[t])
            t += 1
    return acc + b


def _full(s):
    return pl.BlockSpec(s, lambda i: (0,) * len(s))


def _e1_body(x_ref, w_ref, b_ref, o_ref):
    y = _mm(x_ref[0], w_ref[...]) + b_ref[...]
    o_ref[0] = jnp.maximum(y, 0.0)


def _e1_call(xim, w48, b):
    M = H1 * H1
    MB = M // 4
    return pl.pallas_call(
        _e1_body,
        grid=(B, 4),
        in_specs=[pl.BlockSpec((1, MB, 48), lambda i, j: (i, j, 0)),
                  pl.BlockSpec((48, NH // 2), lambda i, j: (0, 0)),
                  pl.BlockSpec((1, NH // 2), lambda i, j: (0, 0))],
        out_specs=pl.BlockSpec((1, MB, NH // 2), lambda i, j: (i, j, 0)),
        out_shape=jax.ShapeDtypeStruct((B, M, NH // 2), jnp.float32),
        compiler_params=pltpu.CompilerParams(
            dimension_semantics=("parallel", "parallel")),
    )(xim, w48, b)


def _e2_body(x_ref, w_ref, b_ref, o_ref):
    y = _mm(x_ref[0], w_ref[...]) + b_ref[...]
    o_ref[0] = jnp.maximum(y, 0.0)


def _e2_call(xim, w512, b2):
    return pl.pallas_call(
        _e2_body,
        grid=(B,),
        in_specs=[pl.BlockSpec((1, M2, 16 * NH // 2), lambda i: (i, 0, 0)),
                  _full((16 * NH // 2, NH)), _full((1, NH))],
        out_specs=pl.BlockSpec((1, M2, NH), lambda i: (i, 0, 0)),
        out_shape=jax.ShapeDtypeStruct((B, M2, NH), jnp.float32),
        compiler_params=_PAR,
    )(xim, w512, b2)


def _c3_body(h_ref, w_ref, b_ref, o_ref, *, Cin):
    o_ref[0] = _conv3(h_ref[0].reshape(H2, H2, Cin), w_ref[...], b_ref[...])


def _c3_call(h, w9, b, Cin, Cout):
    body = functools.partial(_c3_body, Cin=Cin)
    return pl.pallas_call(
        body,
        grid=(B,),
        in_specs=[pl.BlockSpec((1, M2, Cin), lambda i: (i, 0, 0)),
                  _full((9, Cin, Cout)), _full((1, Cout))],
        out_specs=pl.BlockSpec((1, M2, Cout), lambda i: (i, 0, 0)),
        out_shape=jax.ShapeDtypeStruct((B, M2, Cout), jnp.float32),
        compiler_params=_PAR,
    )(h, w9, b)


def _res_body(h_ref, w1_ref, b1_ref, w2_ref, b2_ref, o_ref):
    h = h_ref[0]
    t = jnp.maximum(h, 0.0)
    t = _conv3(t.reshape(H2, H2, NH), w1_ref[...], b1_ref[...])
    t = jnp.maximum(t, 0.0)
    o_ref[0] = h + _mm(t, w2_ref[...]) + b2_ref[...]


def _res_call(h, w1, b1, w2, b2):
    return pl.pallas_call(
        _res_body,
        grid=(B,),
        in_specs=[pl.BlockSpec((1, M2, NH), lambda i: (i, 0, 0)),
                  _full((9, NH, RH)), _full((1, RH)),
                  _full((RH, NH)), _full((1, NH))],
        out_specs=pl.BlockSpec((1, M2, NH), lambda i: (i, 0, 0)),
        out_shape=jax.ShapeDtypeStruct((B, M2, NH), jnp.float32),
        compiler_params=_PAR,
    )(h, w1, b1, w2, b2)


def _vq_body(h_ref, pvw_ref, pvb_ref, cb_ref,
             idx_ref, scal_ref, cnt_ref, sq_ref):
    i = pl.program_id(0)
    h = jnp.maximum(h_ref[...], 0.0)
    f = _mm(h, pvw_ref[...]) + pvb_ref[...]
    cb = cb_ref[...]
    f2 = jnp.sum(f * f, axis=1, keepdims=True)
    c2 = jnp.sum(cb * cb, axis=1)
    dist = f2 - 2.0 * _mm(f, cb.T) + c2[None, :]
    idx = jnp.argmin(dist, axis=1).astype(jnp.int32)
    iota = jax.lax.broadcasted_iota(jnp.int32, (VQ_BLK, K), 1)
    onehot = (idx[:, None] == iota).astype(jnp.float32)
    idx_ref[0, 0] = idx
    sq = jnp.sum(jnp.min(dist, axis=1))
    cnt = jnp.sum(onehot, axis=0, keepdims=True)

    @pl.when(i == 0)
    def _():
        cnt_ref[...] = cnt
        sq_ref[0, 0] = sq

    @pl.when(i > 0)
    def _():
        cnt_ref[...] = cnt_ref[...] + cnt
        sq_ref[0, 0] = sq_ref[0, 0] + sq

    @pl.when(i == VQ_STEPS - 1)
    def _():
        n = jnp.float32(NPIX)
        p = cnt_ref[...] / n
        ent = -jnp.sum(p * jnp.log(p + 1e-10))
        loss = (1.0 + CC) * sq_ref[0, 0] / (n * D)
        lane = jax.lax.broadcasted_iota(jnp.int32, (1, 128), 1)
        scal_ref[...] = jnp.where(lane == 0, loss,
                                  jnp.where(lane == 1, jnp.exp(ent), 0.0))


def _vq_call(hflat, pvw, pvb, cb):
    return pl.pallas_call(
        _vq_body,
        grid=(VQ_STEPS,),
        in_specs=[pl.BlockSpec((VQ_BLK, NH), lambda i: (i, 0)),
                  _full((NH, D)), _full((1, D)),
                  pl.BlockSpec((K, D), lambda i: (0, 0))],
        out_specs=[pl.BlockSpec((1, 1, VQ_BLK), lambda i: (i, 0, 0)),
                   pl.BlockSpec((1, 128), lambda i: (0, 0))],
        out_shape=[jax.ShapeDtypeStruct((VQ_STEPS, 1, VQ_BLK), jnp.int32),
                   jax.ShapeDtypeStruct((1, 128), jnp.float32)],
        scratch_shapes=[pltpu.VMEM((1, K), jnp.float32),
                        pltpu.SMEM((1, 1), jnp.float32)],
    )(hflat, pvw, pvb, cb)


def _sc_gather_body(cb_hbm, idx_hbm, out_hbm, idx_v, rows_v, sem):
    wid = lax.axis_index("s") * 2 + lax.axis_index("c")
    base = wid * SC_CHUNK
    pltpu.sync_copy(idx_hbm.at[pl.ds(base, SC_CHUNK)], idx_v)
    pltpu.async_copy(cb_hbm.at[idx_v], rows_v, sem).wait()
    pltpu.sync_copy(rows_v, out_hbm.at[pl.ds(base, SC_CHUNK)])


def _sc_gather(cb128, idx):
    mesh = plsc.VectorSubcoreMesh(core_axis_name="c", subcore_axis_name="s")
    f = functools.partial(
        pl.kernel, mesh=mesh,
        out_type=jax.ShapeDtypeStruct((NPIX, 128), jnp.float32),
        scratch_types=[pltpu.VMEM((SC_CHUNK,), jnp.int32),
                       pltpu.VMEM((SC_CHUNK, 128), jnp.float32),
                       pltpu.SemaphoreType.DMA],
    )(_sc_gather_body)
    return f(cb128, idx)


def _dtc_body(x_ref, w_ref, b_ref, o_ref, *, relu_in, relu_out):
    x = x_ref[0]
    if relu_in:
        x = jnp.maximum(x, 0.0)
    y = _mm(x, w_ref[...]) + b_ref[...]
    if relu_out:
        y = jnp.maximum(y, 0.0)
    o_ref[0] = y


def _dtc_call(xim, w4, b4, Hin, Cin, Cout, relu_in, relu_out, msplit):
    M = Hin * Hin
    MB = M // msplit
    body = functools.partial(_dtc_body, relu_in=relu_in, relu_out=relu_out)
    return pl.pallas_call(
        body,
        grid=(B, msplit),
        in_specs=[pl.BlockSpec((1, MB, 9 * Cin), lambda i, j: (i, j, 0)),
                  pl.BlockSpec((9 * Cin, 4 * Cout), lambda i, j: (0, 0)),
                  pl.BlockSpec((1, 4 * Cout), lambda i, j: (0, 0))],
        out_specs=pl.BlockSpec((1, MB, 4 * Cout), lambda i, j: (i, j, 0)),
        out_shape=jax.ShapeDtypeStruct((B, M, 4 * Cout), jnp.float32),
        compiler_params=pltpu.CompilerParams(
            dimension_semantics=("parallel", "parallel")),
    )(xim, w4, b4)


def _imcol3(h4):
    Bn, H, W, C = h4.shape
    hp = jnp.pad(h4, ((0, 0), (1, 1), (1, 1), (0, 0)))
    cols = [hp[:, di:di + H, dj:dj + W, :] for di in range(3)
            for dj in range(3)]
    return jnp.concatenate(cols, axis=-1).reshape(Bn, H * W, 9 * C)


def _dtc_weights(w, Cout_pad):
    O, I = w.shape[0], w.shape[1]
    w4 = jnp.zeros((9, I, 4, Cout_pad), jnp.float32)
    for r in range(2):
        for s in range(2):
            for a in range(2):
                for bb in range(2):
                    t = (r + a) * 3 + (s + bb)
                    blk = w[:, :, 2 * a + r, 2 * bb + s].T
                    if Cout_pad != O:
                        blk = jnp.pad(blk, ((0, 0), (0, Cout_pad - O)))
                    w4 = w4.at[t, :, r * 2 + s, :].set(blk)
    return w4.reshape(9 * I, 4 * Cout_pad)


def _tap16(w):
    mats = []
    for pp in range(2):
        for q in range(2):
            for a in range(2):
                for bb in range(2):
                    mats.append(w[:, :, 2 * a + pp, 2 * bb + q].T)
    return jnp.stack(mats, axis=0)


def _tap16T(w):
    mats = []
    for r in range(2):
        for s in range(2):
            for a in range(2):
                for bb in range(2):
                    mats.append(w[:, :, 2 * a + r, 2 * bb + s].T)
    return jnp.stack(mats, axis=0)


def _tap9(w):
    return jnp.stack([w[:, :, di, dj].T for di in range(3)
                      for dj in range(3)], axis=0)


def _parity4(h):
    Bn, H, W, C = h.shape
    hp = jnp.pad(h, ((0, 0), (1, 1), (1, 1), (0, 0)))
    s = hp.reshape(Bn, (H + 2) // 2, 2, (W + 2) // 2, 2, C)
    s = jnp.transpose(s, (0, 2, 4, 1, 3, 5))
    return s.reshape(Bn, 4, (H + 2) // 2, (W + 2) // 2, C)


def _interleave(par, Hh, Cc):
    p = par.reshape(B, Hh, Hh, 2, 2, Cc)
    p = jnp.transpose(p, (0, 1, 3, 2, 4, 5))
    return p.reshape(B, 2 * Hh, 2 * Hh, Cc)


def kernel(x, params):
    p = params
    xn = jnp.transpose(x, (0, 2, 3, 1))

    xp = jnp.pad(xn, ((0, 0), (1, 1), (1, 1), (0, 0)))
    sub = xp.reshape(B, 113, 2, 113, 2, IC)
    cols = []
    for pp in range(2):
        for q in range(2):
            for a in range(2):
                for bb in range(2):
                    cols.append(sub[:, a:a + H1, pp, bb:bb + H1, q, :])
    xim = jnp.concatenate(cols, axis=-1).reshape(B, H1 * H1, 16 * IC)
    w48 = _tap16(p['e1w']).reshape(16 * IC, NH // 2)
    h1 = _e1_call(xim, w48, p['e1b'].reshape(1, -1))

    sub1 = _parity4(h1.reshape(B, H1, H1, NH // 2))
    cols = []
    for pq in range(4):
        for a in range(2):
            for bb in range(2):
                cols.append(sub1[:, pq, a:a + H2, bb:bb + H2, :])
    xim2 = jnp.concatenate(cols, axis=-1).reshape(B, M2, 16 * NH // 2)
    w512 = _tap16(p['e2w']).reshape(16 * NH // 2, NH)
    h = _e2_call(xim2, w512, p['e2b'].reshape(1, -1))
    h = _c3_call(h, _tap9(p['e3w']), p['e3b'].reshape(1, -1), NH, NH)
    for l in range(RL):
        h = _res_call(h, _tap9(p[f'er{l}w1']), p[f'er{l}b1'].reshape(1, -1),
                      p[f'er{l}w2'][:, :, 0, 0].T, p[f'er{l}b2'].reshape(1, -1))

    idx3, scal = _vq_call(h.reshape(NPIX, NH),
                          p['pvw'][:, :, 0, 0].T,
                          p['pvb'].reshape(1, -1), p['cb'])
    idx = idx3.reshape(NPIX)
    cb128 = jnp.pad(p['cb'], ((0, 0), (0, 128 - D)))
    qflat = _sc_gather(cb128, idx)[:, :D]
    vq_loss = scal[0, 0]
    perplexity = scal[0, 1]

    h = _c3_call(qflat.reshape(B, M2, D), _tap9(p['d1w']),
                 p['d1b'].reshape(1, -1), D, NH)
    for l in range(RL):
        h = _res_call(h, _tap9(p[f'dr{l}w1']), p[f'dr{l}b1'].reshape(1, -1),
                      p[f'dr{l}w2'][:, :, 0, 0].T, p[f'dr{l}b2'].reshape(1, -1))

    xim_t1 = _imcol3(h.reshape(B, H2, H2, NH))
    wq1 = _dtc_weights(p['dt1w'], NH // 2)
    bq1 = jnp.tile(p['dt1b'], (4,)).reshape(1, 4 * NH // 2)
    par1 = _dtc_call(xim_t1, wq1, bq1, H2, NH, NH // 2,
                     relu_in=True, relu_out=True, msplit=2)
    up1 = _interleave(par1, H2, NH // 2)

    xim_t2 = _imcol3(up1)
    wq2 = _dtc_weights(p['dt2w'], 8)
    bq2 = jnp.tile(jnp.pad(p['dt2b'], (0, 5)), (4,)).reshape(1, 32)
    par2 = _dtc_call(xim_t2, wq2, bq2, H1, NH // 2, 8,
                     relu_in=False, relu_out=False, msplit=8)
    recon = _interleave(par2, H1, 8)[..., :IC]
    x_recon = jnp.transpose(recon, (0, 3, 1, 2))

    return x_recon, vq_loss, perplexity, idx


# speedup vs baseline: 1.1982x; 1.1982x over previous
import functools

import jax
import jax.numpy as jnp
from jax import lax
from jax.experimental import pallas as pl
from jax.experimental.pallas import tpu as pltpu
from jax.experimental.pallas import tpu_sc as plsc

NH = 64
RH = 32
RL = 2
IC = 3
K = 1024
D = 64
CC = 0.25
B = 8
H1 = 112
H2 = 56
M2 = H2 * H2
NPIX = B * M2
VQ_BLK = 1568
VQ_STEPS = NPIX // VQ_BLK
SC_NW = 32
SC_CHUNK = NPIX // SC_NW

_HIGH = jax.lax.Precision.DEFAULT
_PAR = pltpu.CompilerParams(dimension_semantics=("parallel",))


def _mm(a, b):
    return jax.lax.dot_general(a, b, (((1,), (0,)), ((), ())),
                               precision=_HIGH,
                               preferred_element_type=jnp.float32)


def _pad1(h3):
    H, W, C = h3.shape
    zc = jnp.zeros((H, 1, C), h3.dtype)
    hp = jnp.concatenate([zc, h3, zc], axis=1)
    zr = jnp.zeros((1, W + 2, C), h3.dtype)
    return jnp.concatenate([zr, hp, zr], axis=0)


def _conv3(h3, w9, b):
    H, W, C = h3.shape
    O = w9.shape[2]
    hp = _pad1(h3)
    acc = jnp.zeros((H * W, O), jnp.float32)
    t = 0
    for di in range(3):
        for dj in range(3):
            acc = acc + _mm(hp[di:di + H, dj:dj + W, :].reshape(H * W, C), w9[t])
            t += 1
    return acc + b


def _full(s):
    return pl.BlockSpec(s, lambda i: (0,) * len(s))


def _e1_body(x_ref, w_ref, b_ref, o_ref):
    y = _mm(x_ref[0], w_ref[...]) + b_ref[...]
    o_ref[0] = jnp.maximum(y, 0.0)


def _e1_call(xim, w48, b):
    M = H1 * H1
    MB = M // 4
    return pl.pallas_call(
        _e1_body,
        grid=(B, 4),
        in_specs=[pl.BlockSpec((1, MB, 48), lambda i, j: (i, j, 0)),
                  pl.BlockSpec((48, NH // 2), lambda i, j: (0, 0)),
                  pl.BlockSpec((1, NH // 2), lambda i, j: (0, 0))],
        out_specs=pl.BlockSpec((1, MB, NH // 2), lambda i, j: (i, j, 0)),
        out_shape=jax.ShapeDtypeStruct((B, M, NH // 2), jnp.float32),
        compiler_params=pltpu.CompilerParams(
            dimension_semantics=("parallel", "parallel")),
    )(xim, w48, b)


def _e2_body(x_ref, w_ref, b_ref, o_ref):
    y = _mm(x_ref[0], w_ref[...]) + b_ref[...]
    o_ref[0] = jnp.maximum(y, 0.0)


def _e2_call(xim, w512, b2):
    return pl.pallas_call(
        _e2_body,
        grid=(B,),
        in_specs=[pl.BlockSpec((1, M2, 16 * NH // 2), lambda i: (i, 0, 0)),
                  _full((16 * NH // 2, NH)), _full((1, NH))],
        out_specs=pl.BlockSpec((1, M2, NH), lambda i: (i, 0, 0)),
        out_shape=jax.ShapeDtypeStruct((B, M2, NH), jnp.float32),
        compiler_params=_PAR,
    )(xim, w512, b2)


def _c3_body(h_ref, w_ref, b_ref, o_ref, *, Cin):
    o_ref[0] = _conv3(h_ref[0].reshape(H2, H2, Cin), w_ref[...], b_ref[...])


def _c3_call(h, w9, b, Cin, Cout):
    body = functools.partial(_c3_body, Cin=Cin)
    return pl.pallas_call(
        body,
        grid=(B,),
        in_specs=[pl.BlockSpec((1, M2, Cin), lambda i: (i, 0, 0)),
                  _full((9, Cin, Cout)), _full((1, Cout))],
        out_specs=pl.BlockSpec((1, M2, Cout), lambda i: (i, 0, 0)),
        out_shape=jax.ShapeDtypeStruct((B, M2, Cout), jnp.float32),
        compiler_params=_PAR,
    )(h, w9, b)


def _res_body(h_ref, w1_ref, b1_ref, w2_ref, b2_ref, o_ref):
    h = h_ref[0]
    t = jnp.maximum(h, 0.0)
    t = _conv3(t.reshape(H2, H2, NH), w1_ref[...], b1_ref[...])
    t = jnp.maximum(t, 0.0)
    o_ref[0] = h + _mm(t, w2_ref[...]) + b2_ref[...]


def _res_call(h, w1, b1, w2, b2):
    return pl.pallas_call(
        _res_body,
        grid=(B,),
        in_specs=[pl.BlockSpec((1, M2, NH), lambda i: (i, 0, 0)),
                  _full((9, NH, RH)), _full((1, RH)),
                  _full((RH, NH)), _full((1, NH))],
        out_specs=pl.BlockSpec((1, M2, NH), lambda i: (i, 0, 0)),
        out_shape=jax.ShapeDtypeStruct((B, M2, NH), jnp.float32),
        compiler_params=_PAR,
    )(h, w1, b1, w2, b2)


def _vq_body(h_ref, pvw_ref, pvb_ref, cb_ref,
             idx_ref, scal_ref, cnt_ref, sq_ref):
    i = pl.program_id(0)
    h = jnp.maximum(h_ref[...], 0.0)
    f = _mm(h, pvw_ref[...]) + pvb_ref[...]
    cb = cb_ref[...]
    f2 = jnp.sum(f * f, axis=1, keepdims=True)
    c2 = jnp.sum(cb * cb, axis=1)
    dist = f2 - 2.0 * _mm(f, cb.T) + c2[None, :]
    idx = jnp.argmin(dist, axis=1).astype(jnp.int32)
    iota = jax.lax.broadcasted_iota(jnp.int32, (VQ_BLK, K), 1)
    onehot = (idx[:, None] == iota).astype(jnp.float32)
    idx_ref[0, 0] = idx
    sq = jnp.sum(jnp.min(dist, axis=1))
    cnt = jnp.sum(onehot, axis=0, keepdims=True)

    @pl.when(i == 0)
    def _():
        cnt_ref[...] = cnt
        sq_ref[0, 0] = sq

    @pl.when(i > 0)
    def _():
        cnt_ref[...] = cnt_ref[...] + cnt
        sq_ref[0, 0] = sq_ref[0, 0] + sq

    @pl.when(i == VQ_STEPS - 1)
    def _():
        n = jnp.float32(NPIX)
        p = cnt_ref[...] / n
        ent = -jnp.sum(p * jnp.log(p + 1e-10))
        loss = (1.0 + CC) * sq_ref[0, 0] / (n * D)
        lane = jax.lax.broadcasted_iota(jnp.int32, (1, 128), 1)
        scal_ref[...] = jnp.where(lane == 0, loss,
                                  jnp.where(lane == 1, jnp.exp(ent), 0.0))


def _vq_call(hflat, pvw, pvb, cb):
    return pl.pallas_call(
        _vq_body,
        grid=(VQ_STEPS,),
        in_specs=[pl.BlockSpec((VQ_BLK, NH), lambda i: (i, 0)),
                  _full((NH, D)), _full((1, D)),
                  pl.BlockSpec((K, D), lambda i: (0, 0))],
        out_specs=[pl.BlockSpec((1, 1, VQ_BLK), lambda i: (i, 0, 0)),
                   pl.BlockSpec((1, 128), lambda i: (0, 0))],
        out_shape=[jax.ShapeDtypeStruct((VQ_STEPS, 1, VQ_BLK), jnp.int32),
                   jax.ShapeDtypeStruct((1, 128), jnp.float32)],
        scratch_shapes=[pltpu.VMEM((1, K), jnp.float32),
                        pltpu.SMEM((1, 1), jnp.float32)],
    )(hflat, pvw, pvb, cb)


def _sc_gather_body(cb_hbm, idx_hbm, out_hbm, idx_v, rows_v, sem):
    wid = lax.axis_index("s") * 2 + lax.axis_index("c")
    base = wid * SC_CHUNK
    pltpu.sync_copy(idx_hbm.at[pl.ds(base, SC_CHUNK)], idx_v)
    pltpu.async_copy(cb_hbm.at[idx_v], rows_v, sem).wait()
    pltpu.sync_copy(rows_v, out_hbm.at[pl.ds(base, SC_CHUNK)])


def _sc_gather(cb128, idx):
    mesh = plsc.VectorSubcoreMesh(core_axis_name="c", subcore_axis_name="s")
    f = functools.partial(
        pl.kernel, mesh=mesh,
        out_type=jax.ShapeDtypeStruct((NPIX, 128), jnp.float32),
        scratch_types=[pltpu.VMEM((SC_CHUNK,), jnp.int32),
                       pltpu.VMEM((SC_CHUNK, 128), jnp.float32),
                       pltpu.SemaphoreType.DMA],
    )(_sc_gather_body)
    return f(cb128, idx)


def _dtc_body(x_ref, w_ref, b_ref, o_ref, *, relu_in, relu_out):
    x = x_ref[0]
    if relu_in:
        x = jnp.maximum(x, 0.0)
    y = _mm(x, w_ref[...]) + b_ref[...]
    if relu_out:
        y = jnp.maximum(y, 0.0)
    o_ref[0] = y


def _dtc_call(xim, w4, b4, Hin, Cin, Cout, relu_in, relu_out, msplit):
    M = Hin * Hin
    MB = M // msplit
    body = functools.partial(_dtc_body, relu_in=relu_in, relu_out=relu_out)
    return pl.pallas_call(
        body,
        grid=(B, msplit),
        in_specs=[pl.BlockSpec((1, MB, 9 * Cin), lambda i, j: (i, j, 0)),
                  pl.BlockSpec((9 * Cin, 4 * Cout), lambda i, j: (0, 0)),
                  pl.BlockSpec((1, 4 * Cout), lambda i, j: (0, 0))],
        out_specs=pl.BlockSpec((1, MB, 4 * Cout), lambda i, j: (i, j, 0)),
        out_shape=jax.ShapeDtypeStruct((B, M, 4 * Cout), jnp.float32),
        compiler_params=pltpu.CompilerParams(
            dimension_semantics=("parallel", "parallel")),
    )(xim, w4, b4)


def _dt2_body(hp_ref, w_ref, b_ref, o_ref):
    j = pl.program_id(1)
    RB = H1 // 8
    acc = jnp.zeros((RB * H1, 32), jnp.float32)
    t = 0
    for di in range(3):
        for dj in range(3):
            x = hp_ref[0, pl.ds(RB * j + di, RB), pl.ds(dj, H1), :]
            acc = acc + _mm(x.reshape(RB * H1, NH // 2), w_ref[t])
            t += 1
    o_ref[0] = acc + b_ref[...]


def _dt2_call(hp, w9, b4):
    RB = H1 // 8
    return pl.pallas_call(
        _dt2_body,
        grid=(B, 8),
        in_specs=[pl.BlockSpec((1, H1 + 2, H1 + 2, NH // 2),
                               lambda i, j: (i, 0, 0, 0)),
                  pl.BlockSpec((9, NH // 2, 32), lambda i, j: (0, 0, 0)),
                  pl.BlockSpec((1, 32), lambda i, j: (0, 0))],
        out_specs=pl.BlockSpec((1, RB * H1, 32), lambda i, j: (i, j, 0)),
        out_shape=jax.ShapeDtypeStruct((B, H1 * H1, 32), jnp.float32),
        compiler_params=pltpu.CompilerParams(
            dimension_semantics=("parallel", "arbitrary")),
    )(hp, w9, b4)


def _imcol3(h4):
    Bn, H, W, C = h4.shape
    hp = jnp.pad(h4, ((0, 0), (1, 1), (1, 1), (0, 0)))
    cols = [hp[:, di:di + H, dj:dj + W, :] for di in range(3)
            for dj in range(3)]
    return jnp.concatenate(cols, axis=-1).reshape(Bn, H * W, 9 * C)


def _dtc_weights(w, Cout_pad):
    O, I = w.shape[0], w.shape[1]
    w4 = jnp.zeros((9, I, 4, Cout_pad), jnp.float32)
    for r in range(2):
        for s in range(2):
            for a in range(2):
                for bb in range(2):
                    t = (r + a) * 3 + (s + bb)
                    blk = w[:, :, 2 * a + r, 2 * bb + s].T
                    if Cout_pad != O:
                        blk = jnp.pad(blk, ((0, 0), (0, Cout_pad - O)))
                    w4 = w4.at[t, :, r * 2 + s, :].set(blk)
    return w4.reshape(9 * I, 4 * Cout_pad)


def _tap16(w):
    mats = []
    for pp in range(2):
        for q in range(2):
            for a in range(2):
                for bb in range(2):
                    mats.append(w[:, :, 2 * a + pp, 2 * bb + q].T)
    return jnp.stack(mats, axis=0)


def _tap9(w):
    return jnp.stack([w[:, :, di, dj].T for di in range(3)
                      for dj in range(3)], axis=0)


def _parity4(h):
    Bn, H, W, C = h.shape
    hp = jnp.pad(h, ((0, 0), (1, 1), (1, 1), (0, 0)))
    s = hp.reshape(Bn, (H + 2) // 2, 2, (W + 2) // 2, 2, C)
    s = jnp.transpose(s, (0, 2, 4, 1, 3, 5))
    return s.reshape(Bn, 4, (H + 2) // 2, (W + 2) // 2, C)


def _interleave(par, Hh, Cc):
    p = par.reshape(B, Hh, Hh, 2, 2, Cc)
    p = jnp.transpose(p, (0, 1, 3, 2, 4, 5))
    return p.reshape(B, 2 * Hh, 2 * Hh, Cc)


def kernel(x, params):
    p = params
    xn = jnp.transpose(x, (0, 2, 3, 1))

    xp = jnp.pad(xn, ((0, 0), (1, 1), (1, 1), (0, 0)))
    sub = xp.reshape(B, 113, 2, 113, 2, IC)
    cols = []
    for pp in range(2):
        for q in range(2):
            for a in range(2):
                for bb in range(2):
                    cols.append(sub[:, a:a + H1, pp, bb:bb + H1, q, :])
    xim = jnp.concatenate(cols, axis=-1).reshape(B, H1 * H1, 16 * IC)
    w48 = _tap16(p['e1w']).reshape(16 * IC, NH // 2)
    h1 = _e1_call(xim, w48, p['e1b'].reshape(1, -1))

    sub1 = _parity4(h1.reshape(B, H1, H1, NH // 2))
    cols = []
    for pq in range(4):
        for a in range(2):
            for bb in range(2):
                cols.append(sub1[:, pq, a:a + H2, bb:bb + H2, :])
    xim2 = jnp.concatenate(cols, axis=-1).reshape(B, M2, 16 * NH // 2)
    w512 = _tap16(p['e2w']).reshape(16 * NH // 2, NH)
    h = _e2_call(xim2, w512, p['e2b'].reshape(1, -1))
    h = _c3_call(h, _tap9(p['e3w']), p['e3b'].reshape(1, -1), NH, NH)
    for l in range(RL):
        h = _res_call(h, _tap9(p[f'er{l}w1']), p[f'er{l}b1'].reshape(1, -1),
                      p[f'er{l}w2'][:, :, 0, 0].T, p[f'er{l}b2'].reshape(1, -1))

    idx3, scal = _vq_call(h.reshape(NPIX, NH),
                          p['pvw'][:, :, 0, 0].T,
                          p['pvb'].reshape(1, -1), p['cb'])
    idx = idx3.reshape(NPIX)
    cb128 = jnp.pad(p['cb'], ((0, 0), (0, 128 - D)))
    qflat = _sc_gather(cb128, idx)[:, :D]
    vq_loss = scal[0, 0]
    perplexity = scal[0, 1]

    h = _c3_call(qflat.reshape(B, M2, D), _tap9(p['d1w']),
                 p['d1b'].reshape(1, -1), D, NH)
    for l in range(RL):
        h = _res_call(h, _tap9(p[f'dr{l}w1']), p[f'dr{l}b1'].reshape(1, -1),
                      p[f'dr{l}w2'][:, :, 0, 0].T, p[f'dr{l}b2'].reshape(1, -1))

    xim_t1 = _imcol3(h.reshape(B, H2, H2, NH))
    wq1 = _dtc_weights(p['dt1w'], NH // 2)
    bq1 = jnp.tile(p['dt1b'], (4,)).reshape(1, 4 * NH // 2)
    par1 = _dtc_call(xim_t1, wq1, bq1, H2, NH, NH // 2,
                     relu_in=True, relu_out=True, msplit=2)
    up1 = _interleave(par1, H2, NH // 2)

    hp2 = jnp.pad(up1, ((0, 0), (1, 1), (1, 1), (0, 0)))
    wq2 = _dtc_weights(p['dt2w'], 8).reshape(9, NH // 2, 32)
    bq2 = jnp.tile(jnp.pad(p['dt2b'], (0, 5)), (4,)).reshape(1, 32)
    par2 = _dt2_call(hp2, wq2, bq2)
    recon = _interleave(par2, H1, 8)[..., :IC]
    x_recon = jnp.transpose(recon, (0, 3, 1, 2))

    return x_recon, vq_loss, perplexity, idx


# speedup vs baseline: 1.2171x; 1.0158x over previous
import functools

import jax
import jax.numpy as jnp
from jax import lax
from jax.experimental import pallas as pl
from jax.experimental.pallas import tpu as pltpu
from jax.experimental.pallas import tpu_sc as plsc

NH = 64
RH = 32
RL = 2
IC = 3
K = 1024
D = 64
CC = 0.25
B = 8
H1 = 112
H2 = 56
M2 = H2 * H2
NPIX = B * M2
VQ_BLK = 1568
VQ_STEPS = NPIX // VQ_BLK
SC_NW = 32
SC_CHUNK = NPIX // SC_NW

_HIGH = jax.lax.Precision.DEFAULT
_PAR = pltpu.CompilerParams(dimension_semantics=("parallel",))


def _mm(a, b):
    return jax.lax.dot_general(a, b, (((1,), (0,)), ((), ())),
                               precision=_HIGH,
                               preferred_element_type=jnp.float32)


def _pad1(h3):
    H, W, C = h3.shape
    zc = jnp.zeros((H, 1, C), h3.dtype)
    hp = jnp.concatenate([zc, h3, zc], axis=1)
    zr = jnp.zeros((1, W + 2, C), h3.dtype)
    return jnp.concatenate([zr, hp, zr], axis=0)


def _conv3(h3, w9, b):
    H, W, C = h3.shape
    O = w9.shape[2]
    hp = _pad1(h3)
    acc = jnp.zeros((H * W, O), jnp.float32)
    t = 0
    for di in range(3):
        for dj in range(3):
            acc = acc + _mm(hp[di:di + H, dj:dj + W, :].reshape(H * W, C), w9[t])
            t += 1
    return acc + b


def _full(s):
    return pl.BlockSpec(s, lambda i: (0,) * len(s))


def _e1_body(x_ref, w_ref, b_ref, o_ref):
    y = _mm(x_ref[0], w_ref[...]) + b_ref[...]
    o_ref[0] = jnp.maximum(y, 0.0)


def _e1_call(xim, w48, b):
    M = H1 * H1
    MB = M // 4
    return pl.pallas_call(
        _e1_body,
        grid=(B, 4),
        in_specs=[pl.BlockSpec((1, MB, 48), lambda i, j: (i, j, 0)),
                  pl.BlockSpec((48, NH // 2), lambda i, j: (0, 0)),
                  pl.BlockSpec((1, NH // 2), lambda i, j: (0, 0))],
        out_specs=pl.BlockSpec((1, MB, NH // 2), lambda i, j: (i, j, 0)),
        out_shape=jax.ShapeDtypeStruct((B, M, NH // 2), jnp.float32),
        compiler_params=pltpu.CompilerParams(
            dimension_semantics=("parallel", "parallel")),
    )(xim, w48, b)


def _e2_body(x_ref, w_ref, b_ref, o_ref):
    y = _mm(x_ref[0], w_ref[...]) + b_ref[...]
    o_ref[0] = jnp.maximum(y, 0.0)


def _e2_call(xim, w512, b2):
    return pl.pallas_call(
        _e2_body,
        grid=(B,),
        in_specs=[pl.BlockSpec((1, M2, 16 * NH // 2), lambda i: (i, 0, 0)),
                  _full((16 * NH // 2, NH)), _full((1, NH))],
        out_specs=pl.BlockSpec((1, M2, NH), lambda i: (i, 0, 0)),
        out_shape=jax.ShapeDtypeStruct((B, M2, NH), jnp.float32),
        compiler_params=_PAR,
    )(xim, w512, b2)


def _c3_body(hp_ref, w_ref, b_ref, o_ref, *, Cin):
    O = w_ref.shape[2]
    acc = jnp.zeros((M2, O), jnp.float32)
    t = 0
    for di in range(3):
        for dj in range(3):
            x = hp_ref[0, di:di + H2, dj:dj + H2, :].reshape(M2, Cin)
            acc = acc + _mm(x, w_ref[t])
            t += 1
    o_ref[0] = acc + b_ref[...]


def _c3_call(h4, w9, b, Cin, Cout):
    hp = jnp.pad(h4, ((0, 0), (1, 1), (1, 1), (0, 0)))
    body = functools.partial(_c3_body, Cin=Cin)
    return pl.pallas_call(
        body,
        grid=(B,),
        in_specs=[pl.BlockSpec((1, H2 + 2, H2 + 2, Cin),
                               lambda i: (i, 0, 0, 0)),
                  _full((9, Cin, Cout)), _full((1, Cout))],
        out_specs=pl.BlockSpec((1, M2, Cout), lambda i: (i, 0, 0)),
        out_shape=jax.ShapeDtypeStruct((B, M2, Cout), jnp.float32),
        compiler_params=_PAR,
    )(hp, w9, b)


def _res_body(hp_ref, w1_ref, b1_ref, w2_ref, b2_ref, o_ref):
    acc = jnp.zeros((M2, RH), jnp.float32)
    t = 0
    for di in range(3):
        for dj in range(3):
            x = hp_ref[0, di:di + H2, dj:dj + H2, :].reshape(M2, NH)
            acc = acc + _mm(jnp.maximum(x, 0.0), w1_ref[t])
            t += 1
    tt = jnp.maximum(acc + b1_ref[...], 0.0)
    h = hp_ref[0, 1:H2 + 1, 1:H2 + 1, :].reshape(M2, NH)
    o_ref[0] = h + _mm(tt, w2_ref[...]) + b2_ref[...]


def _res_call(h4, w1, b1, w2, b2):
    hp = jnp.pad(h4, ((0, 0), (1, 1), (1, 1), (0, 0)))
    return pl.pallas_call(
        _res_body,
        grid=(B,),
        in_specs=[pl.BlockSpec((1, H2 + 2, H2 + 2, NH),
                               lambda i: (i, 0, 0, 0)),
                  _full((9, NH, RH)), _full((1, RH)),
                  _full((RH, NH)), _full((1, NH))],
        out_specs=pl.BlockSpec((1, M2, NH), lambda i: (i, 0, 0)),
        out_shape=jax.ShapeDtypeStruct((B, M2, NH), jnp.float32),
        compiler_params=_PAR,
    )(hp, w1, b1, w2, b2)


def _vq_body(h_ref, pvw_ref, pvb_ref, cb_ref,
             idx_ref, scal_ref, cnt_ref, sq_ref):
    i = pl.program_id(0)
    h = jnp.maximum(h_ref[...], 0.0)
    f = _mm(h, pvw_ref[...]) + pvb_ref[...]
    cb = cb_ref[...]
    f2 = jnp.sum(f * f, axis=1, keepdims=True)
    c2 = jnp.sum(cb * cb, axis=1)
    dist = f2 - 2.0 * _mm(f, cb.T) + c2[None, :]
    idx = jnp.argmin(dist, axis=1).astype(jnp.int32)
    iota = jax.lax.broadcasted_iota(jnp.int32, (VQ_BLK, K), 1)
    onehot = (idx[:, None] == iota).astype(jnp.float32)
    idx_ref[0, 0] = idx
    sq = jnp.sum(jnp.min(dist, axis=1))
    cnt = jnp.sum(onehot, axis=0, keepdims=True)

    @pl.when(i == 0)
    def _():
        cnt_ref[...] = cnt
        sq_ref[0, 0] = sq

    @pl.when(i > 0)
    def _():
        cnt_ref[...] = cnt_ref[...] + cnt
        sq_ref[0, 0] = sq_ref[0, 0] + sq

    @pl.when(i == VQ_STEPS - 1)
    def _():
        n = jnp.float32(NPIX)
        p = cnt_ref[...] / n
        ent = -jnp.sum(p * jnp.log(p + 1e-10))
        loss = (1.0 + CC) * sq_ref[0, 0] / (n * D)
        lane = jax.lax.broadcasted_iota(jnp.int32, (1, 128), 1)
        scal_ref[...] = jnp.where(lane == 0, loss,
                                  jnp.where(lane == 1, jnp.exp(ent), 0.0))


def _vq_call(hflat, pvw, pvb, cb):
    return pl.pallas_call(
        _vq_body,
        grid=(VQ_STEPS,),
        in_specs=[pl.BlockSpec((VQ_BLK, NH), lambda i: (i, 0)),
                  _full((NH, D)), _full((1, D)),
                  pl.BlockSpec((K, D), lambda i: (0, 0))],
        out_specs=[pl.BlockSpec((1, 1, VQ_BLK), lambda i: (i, 0, 0)),
                   pl.BlockSpec((1, 128), lambda i: (0, 0))],
        out_shape=[jax.ShapeDtypeStruct((VQ_STEPS, 1, VQ_BLK), jnp.int32),
                   jax.ShapeDtypeStruct((1, 128), jnp.float32)],
        scratch_shapes=[pltpu.VMEM((1, K), jnp.float32),
                        pltpu.SMEM((1, 1), jnp.float32)],
    )(hflat, pvw, pvb, cb)


def _sc_gather_body(cb_hbm, idx_hbm, out_hbm, idx_v, rows_v, sem):
    wid = lax.axis_index("s") * 2 + lax.axis_index("c")
    base = wid * SC_CHUNK
    pltpu.sync_copy(idx_hbm.at[pl.ds(base, SC_CHUNK)], idx_v)
    pltpu.async_copy(cb_hbm.at[idx_v], rows_v, sem).wait()
    pltpu.sync_copy(rows_v, out_hbm.at[pl.ds(base, SC_CHUNK)])


def _sc_gather(cb128, idx):
    mesh = plsc.VectorSubcoreMesh(core_axis_name="c", subcore_axis_name="s")
    f = functools.partial(
        pl.kernel, mesh=mesh,
        out_type=jax.ShapeDtypeStruct((NPIX, 128), jnp.float32),
        scratch_types=[pltpu.VMEM((SC_CHUNK,), jnp.int32),
                       pltpu.VMEM((SC_CHUNK, 128), jnp.float32),
                       pltpu.SemaphoreType.DMA],
    )(_sc_gather_body)
    return f(cb128, idx)


def _dtc_body(x_ref, w_ref, b_ref, o_ref, *, relu_in, relu_out):
    x = x_ref[0]
    if relu_in:
        x = jnp.maximum(x, 0.0)
    y = _mm(x, w_ref[...]) + b_ref[...]
    if relu_out:
        y = jnp.maximum(y, 0.0)
    o_ref[0] = y


def _dtc_call(xim, w4, b4, Hin, Cin, Cout, relu_in, relu_out, msplit):
    M = Hin * Hin
    MB = M // msplit
    body = functools.partial(_dtc_body, relu_in=relu_in, relu_out=relu_out)
    return pl.pallas_call(
        body,
        grid=(B, msplit),
        in_specs=[pl.BlockSpec((1, MB, 9 * Cin), lambda i, j: (i, j, 0)),
                  pl.BlockSpec((9 * Cin, 4 * Cout), lambda i, j: (0, 0)),
                  pl.BlockSpec((1, 4 * Cout), lambda i, j: (0, 0))],
        out_specs=pl.BlockSpec((1, MB, 4 * Cout), lambda i, j: (i, j, 0)),
        out_shape=jax.ShapeDtypeStruct((B, M, 4 * Cout), jnp.float32),
        compiler_params=pltpu.CompilerParams(
            dimension_semantics=("parallel", "parallel")),
    )(xim, w4, b4)


def _dt1_body(hp_ref, w_ref, b_ref, o_ref):
    j = pl.program_id(1)
    RB = H2 // 4
    acc = jnp.zeros((RB * H2, 4 * (NH // 2)), jnp.float32)
    t = 0
    for di in range(3):
        for dj in range(3):
            x = hp_ref[0, pl.ds(RB * j + di, RB), pl.ds(dj, H2), :]
            x = jnp.maximum(x.reshape(RB * H2, NH), 0.0)
            acc = acc + _mm(x, w_ref[t])
            t += 1
    o_ref[0] = jnp.maximum(acc + b_ref[...], 0.0)


def _dt1_call(hp, w9, b4):
    RB = H2 // 4
    return pl.pallas_call(
        _dt1_body,
        grid=(B, 4),
        in_specs=[pl.BlockSpec((1, H2 + 2, H2 + 2, NH),
                               lambda i, j: (i, 0, 0, 0)),
                  pl.BlockSpec((9, NH, 4 * (NH // 2)), lambda i, j: (0, 0, 0)),
                  pl.BlockSpec((1, 4 * (NH // 2)), lambda i, j: (0, 0))],
        out_specs=pl.BlockSpec((1, RB * H2, 4 * (NH // 2)),
                               lambda i, j: (i, j, 0)),
        out_shape=jax.ShapeDtypeStruct((B, M2, 4 * (NH // 2)), jnp.float32),
        compiler_params=pltpu.CompilerParams(
            dimension_semantics=("parallel", "arbitrary")),
    )(hp, w9, b4)


def _dt2_body(hp_ref, w_ref, b_ref, o_ref):
    j = pl.program_id(1)
    RB = H1 // 8
    acc = jnp.zeros((RB * H1, 32), jnp.float32)
    t = 0
    for di in range(3):
        for dj in range(3):
            x = hp_ref[0, pl.ds(RB * j + di, RB), pl.ds(dj, H1), :]
            acc = acc + _mm(x.reshape(RB * H1, NH // 2), w_ref[t])
            t += 1
    o_ref[0] = acc + b_ref[...]


def _dt2_call(hp, w9, b4):
    RB = H1 // 8
    return pl.pallas_call(
        _dt2_body,
        grid=(B, 8),
        in_specs=[pl.BlockSpec((1, H1 + 2, H1 + 2, NH // 2),
                               lambda i, j: (i, 0, 0, 0)),
                  pl.BlockSpec((9, NH // 2, 32), lambda i, j: (0, 0, 0)),
                  pl.BlockSpec((1, 32), lambda i, j: (0, 0))],
        out_specs=pl.BlockSpec((1, RB * H1, 32), lambda i, j: (i, j, 0)),
        out_shape=jax.ShapeDtypeStruct((B, H1 * H1, 32), jnp.float32),
        compiler_params=pltpu.CompilerParams(
            dimension_semantics=("parallel", "arbitrary")),
    )(hp, w9, b4)


def _imcol3(h4):
    Bn, H, W, C = h4.shape
    hp = jnp.pad(h4, ((0, 0), (1, 1), (1, 1), (0, 0)))
    cols = [hp[:, di:di + H, dj:dj + W, :] for di in range(3)
            for dj in range(3)]
    return jnp.concatenate(cols, axis=-1).reshape(Bn, H * W, 9 * C)


def _dtc_weights(w, Cout_pad):
    O, I = w.shape[0], w.shape[1]
    w4 = jnp.zeros((9, I, 4, Cout_pad), jnp.float32)
    for r in range(2):
        for s in range(2):
            for a in range(2):
                for bb in range(2):
                    t = (r + a) * 3 + (s + bb)
                    blk = w[:, :, 2 * a + r, 2 * bb + s].T
                    if Cout_pad != O:
                        blk = jnp.pad(blk, ((0, 0), (0, Cout_pad - O)))
                    w4 = w4.at[t, :, r * 2 + s, :].set(blk)
    return w4.reshape(9 * I, 4 * Cout_pad)


def _tap16(w):
    mats = []
    for pp in range(2):
        for q in range(2):
            for a in range(2):
                for bb in range(2):
                    mats.append(w[:, :, 2 * a + pp, 2 * bb + q].T)
    return jnp.stack(mats, axis=0)


def _tap9(w):
    return jnp.stack([w[:, :, di, dj].T for di in range(3)
                      for dj in range(3)], axis=0)


def _parity4(h):
    Bn, H, W, C = h.shape
    hp = jnp.pad(h, ((0, 0), (1, 1), (1, 1), (0, 0)))
    s = hp.reshape(Bn, (H + 2) // 2, 2, (W + 2) // 2, 2, C)
    s = jnp.transpose(s, (0, 2, 4, 1, 3, 5))
    return s.reshape(Bn, 4, (H + 2) // 2, (W + 2) // 2, C)


def _interleave(par, Hh, Cc):
    p = par.reshape(B, Hh, Hh, 2, 2, Cc)
    p = jnp.transpose(p, (0, 1, 3, 2, 4, 5))
    return p.reshape(B, 2 * Hh, 2 * Hh, Cc)


def kernel(x, params):
    p = params
    xn = jnp.transpose(x, (0, 2, 3, 1))

    xp = jnp.pad(xn, ((0, 0), (1, 1), (1, 1), (0, 0)))
    sub = xp.reshape(B, 113, 2, 113, 2, IC)
    cols = []
    for pp in range(2):
        for q in range(2):
            for a in range(2):
                for bb in range(2):
                    cols.append(sub[:, a:a + H1, pp, bb:bb + H1, q, :])
    xim = jnp.concatenate(cols, axis=-1).reshape(B, H1 * H1, 16 * IC)
    w48 = _tap16(p['e1w']).reshape(16 * IC, NH // 2)
    h1 = _e1_call(xim, w48, p['e1b'].reshape(1, -1))

    sub1 = _parity4(h1.reshape(B, H1, H1, NH // 2))
    cols = []
    for pq in range(4):
        for a in range(2):
            for bb in range(2):
                cols.append(sub1[:, pq, a:a + H2, bb:bb + H2, :])
    xim2 = jnp.concatenate(cols, axis=-1).reshape(B, M2, 16 * NH // 2)
    w512 = _tap16(p['e2w']).reshape(16 * NH // 2, NH)
    h = _e2_call(xim2, w512, p['e2b'].reshape(1, -1))
    h = _c3_call(h.reshape(B, H2, H2, NH), _tap9(p['e3w']),
                 p['e3b'].reshape(1, -1), NH, NH)
    for l in range(RL):
        h = _res_call(h.reshape(B, H2, H2, NH), _tap9(p[f'er{l}w1']),
                      p[f'er{l}b1'].reshape(1, -1),
                      p[f'er{l}w2'][:, :, 0, 0].T, p[f'er{l}b2'].reshape(1, -1))

    idx3, scal = _vq_call(h.reshape(NPIX, NH),
                          p['pvw'][:, :, 0, 0].T,
                          p['pvb'].reshape(1, -1), p['cb'])
    idx = idx3.reshape(NPIX)
    cb128 = jnp.pad(p['cb'], ((0, 0), (0, 128 - D)))
    qflat = _sc_gather(cb128, idx)[:, :D]
    vq_loss = scal[0, 0]
    perplexity = scal[0, 1]

    h = _c3_call(qflat.reshape(B, H2, H2, D), _tap9(p['d1w']),
                 p['d1b'].reshape(1, -1), D, NH)
    for l in range(RL):
        h = _res_call(h.reshape(B, H2, H2, NH), _tap9(p[f'dr{l}w1']),
                      p[f'dr{l}b1'].reshape(1, -1),
                      p[f'dr{l}w2'][:, :, 0, 0].T, p[f'dr{l}b2'].reshape(1, -1))

    hp1 = jnp.pad(h.reshape(B, H2, H2, NH),
                  ((0, 0), (1, 1), (1, 1), (0, 0)))
    wq1 = _dtc_weights(p['dt1w'], NH // 2).reshape(9, NH, 4 * (NH // 2))
    bq1 = jnp.tile(p['dt1b'], (4,)).reshape(1, 4 * NH // 2)
    par1 = _dt1_call(hp1, wq1, bq1)
    up1 = _interleave(par1, H2, NH // 2)

    hp2 = jnp.pad(up1, ((0, 0), (1, 1), (1, 1), (0, 0)))
    wq2 = _dtc_weights(p['dt2w'], 8).reshape(9, NH // 2, 32)
    bq2 = jnp.tile(jnp.pad(p['dt2b'], (0, 5)), (4,)).reshape(1, 32)
    par2 = _dt2_call(hp2, wq2, bq2)
    recon = _interleave(par2, H1, 8)[..., :IC]
    x_recon = jnp.transpose(recon, (0, 3, 1, 2))

    return x_recon, vq_loss, perplexity, idx
